# baseline (speedup 1.0000x reference)
import functools

import jax
import jax.numpy as jnp
from jax import lax
from jax.experimental import pallas as pl
from jax.experimental.pallas import tpu as pltpu
from jax.experimental.pallas import tpu_sc as plsc

NU = 5000
NN = 10000
NNP = 10112
EDGES = 320000
D = 128
NL = 3
BATCH = 4096

NC = 2
NS = 16
NW = NC * NS
CHUNK = 112
NBUF = 3
IBUF = 6
NCH0 = 126
NCH1 = 54
NCHT = NCH0 + NCH1
EPAD = NCHT * NS * CHUNK
_bf16 = jnp.bfloat16
PPT = BATCH // NW

_f32 = jnp.float32
_mesh = plsc.VectorSubcoreMesh(core_axis_name="c", subcore_axis_name="s")


@functools.partial(
    pl.kernel,
    out_type=jax.ShapeDtypeStruct((NC, NNP, D), _f32),
    mesh=_mesh,
    scratch_types=[
        pltpu.VMEM((IBUF, 2, CHUNK), jnp.int32),
        pltpu.VMEM((IBUF * CHUNK,), _f32),
        [pltpu.VMEM((CHUNK, D), _f32) for _ in range(NBUF)],
        pltpu.VMEM_SHARED((NNP, D), _f32),
        [pltpu.SemaphoreType.DMA for _ in range(NBUF)],
        [pltpu.SemaphoreType.DMA for _ in range(NBUF)],
        [pltpu.SemaphoreType.DMA for _ in range(IBUF)],
    ],
    compiler_params=pltpu.CompilerParams(needs_layout_passes=False),
)
def _sc_scatter(ego_hbm, col_hbm, row_hbm, val_hbm, z_hbm, out_hbm,
                ring, vring, bufs, side_sh, gsems, ssems, isems):
    c = lax.axis_index("c")
    s = lax.axis_index("s")
    nch = jnp.where(c == 0, NCH0, NCH1)
    cbase = jnp.where(c == 0, s * NCH0, NS * NCH0 + s * NCH1)
    stripe = NNP // NS
    pltpu.sync_copy(z_hbm.at[pl.ds(s * stripe, stripe)],
                    side_sh.at[pl.ds(s * stripe, stripe)])

    def idx_load_col(g, ib):
        return pltpu.make_async_copy(
            col_hbm.at[pl.ds((cbase + g) * CHUNK, CHUNK)],
            ring.at[ib, 0], isems[ib])

    def idx_load_row(g, ib):
        return pltpu.make_async_copy(
            row_hbm.at[pl.ds((cbase + g) * CHUNK, CHUNK)],
            ring.at[ib, 1], isems[ib])

    def idx_load_val(g, ib):
        return pltpu.make_async_copy(
            val_hbm.at[pl.ds((cbase + g) * CHUNK, CHUNK)],
            vring.at[pl.ds(ib * CHUNK, CHUNK)], isems[ib])

    def idx_start(g, ib):
        idx_load_col(g, ib).start()
        idx_load_row(g, ib).start()
        idx_load_val(g, ib).start()

    def idx_wait(g, ib):
        idx_load_col(g, ib).wait()
        idx_load_row(g, ib).wait()
        idx_load_val(g, ib).wait()

    def gather(g_dyn, b, ib):
        return pltpu.make_async_copy(ego_hbm.at[ring.at[ib, 0]], bufs[b],
                                     gsems[b])

    def scatter(g_dyn, b, ib):
        return pltpu.make_async_copy(bufs[b], side_sh.at[ring.at[ib, 1]],
                                     ssems[b])

    def scale(g, b, ib):
        buf = bufs[b]

        def do_group(q):
            vv16 = vring[pl.ds(ib * CHUNK + q * 16, 16)]
            for j2 in range(16):
                vv = lax.gather(
                    vv16, jnp.full((16, 1), j2, jnp.int32),
                    lax.GatherDimensionNumbers(
                        offset_dims=(), collapsed_slice_dims=(0,),
                        start_index_map=(0,)),
                    slice_sizes=(1,),
                    mode=lax.GatherScatterMode.PROMISE_IN_BOUNDS)
                for r in range(D // 16):
                    sl = (q * 16 + j2, pl.ds(r * 16, 16))
                    buf[sl] = buf[sl] * vv

        def group2(h, cc):
            for dq in range(2):
                do_group(h * 2 + dq)
            return cc

        lax.fori_loop(0, CHUNK // 32, group2, 0)
        for q in range(CHUNK // 32 * 2, CHUNK // 16):
            do_group(q)

    for k in range(4):
        idx_start(k, k)
    idx_wait(0, 0)
    gather(0, 0, 0).start()

    def pipe(h, carry):
        g0 = h * IBUF
        for j in range(IBUF):
            g = g0 + j
            b = j % NBUF
            nb = (j + 1) % NBUF
            nib = (j + 1) % IBUF
            gather(g, b, j).wait()

            @pl.when(g >= 2)
            def _():
                scatter(g - 2, nb, nib).wait()

            @pl.when(g + 1 < nch)
            def _():
                idx_wait(g + 1, nib)
                gather(g + 1, nb, nib).start()

            scale(g, b, j)
            scatter(g, b, j).start(add=True)

            @pl.when(g + 4 < nch)
            def _():
                idx_start(g + 4, (j + 4) % IBUF)
        return carry

    lax.fori_loop(0, nch // IBUF, pipe, 0)
    scatter(0, (-2) % NBUF, (-2) % IBUF).wait()
    scatter(0, (-1) % NBUF, (-1) % IBUF).wait()
    plsc.subcore_barrier()
    pltpu.sync_copy(side_sh.at[pl.ds(s * stripe, stripe)],
                    out_hbm.at[c, pl.ds(s * stripe, stripe)])


_BLK = 2000


def _layer_body(parts_r, ego_r, wg_r, bg_r, wb_r, bb_r, ego_o, norm_o):
    side = parts_r[0] + parts_r[1]
    e = ego_r[...]
    x = jnp.dot(side, wg_r[...], preferred_element_type=_f32) + bg_r[...]
    se = jnp.where(x >= 0, x, 0.01 * x)
    y = jnp.dot(e * side, wb_r[...], preferred_element_type=_f32) + bb_r[...]
    bi = jnp.where(y >= 0, y, 0.01 * y)
    new = se + bi
    nrm = jnp.sqrt(jnp.sum(new * new, axis=1, keepdims=True))
    ego_o[...] = new
    norm_o[...] = new / jnp.maximum(nrm, 1e-12)


_tc_layer = pl.pallas_call(
    _layer_body,
    grid=(NN // _BLK,),
    in_specs=[
        pl.BlockSpec((NC, _BLK, D), lambda i: (0, i, 0)),
        pl.BlockSpec((_BLK, D), lambda i: (i, 0)),
        pl.BlockSpec((D, D), lambda i: (0, 0)),
        pl.BlockSpec((1, D), lambda i: (0, 0)),
        pl.BlockSpec((D, D), lambda i: (0, 0)),
        pl.BlockSpec((1, D), lambda i: (0, 0)),
    ],
    out_specs=[pl.BlockSpec((_BLK, D), lambda i: (i, 0))] * 2,
    out_shape=[jax.ShapeDtypeStruct((NN, D), _f32)] * 2,
)


@functools.partial(
    pl.kernel,
    out_type=(jax.ShapeDtypeStruct((NL + 1, BATCH, D), _f32),
              jax.ShapeDtypeStruct((NL + 1, BATCH, D), _f32)),
    mesh=_mesh,
    scratch_types=[
        pltpu.VMEM((PPT,), jnp.int32),
        pltpu.VMEM((PPT,), jnp.int32),
        pltpu.VMEM((PPT, D), _f32),
        pltpu.SemaphoreType.DMA,
    ],
)
def _sc_gather(t0, t1, t2, t3, u_hbm, it_hbm, uo_hbm, io_hbm,
               uv, iv, buf, sem):
    c = lax.axis_index("c")
    s = lax.axis_index("s")
    base = (c * NS + s) * PPT
    pltpu.sync_copy(u_hbm.at[pl.ds(base, PPT)], uv)
    pltpu.sync_copy(it_hbm.at[pl.ds(base, PPT)], iv)
    for r in range(PPT // 16):
        iv[pl.ds(r * 16, 16)] = iv[pl.ds(r * 16, 16)] + NU
    for t, tbl in enumerate((t0, t1, t2, t3)):
        pltpu.async_copy(tbl.at[uv], buf, sem).wait()
        pltpu.sync_copy(buf, uo_hbm.at[t, pl.ds(base, PPT)])
        pltpu.async_copy(tbl.at[iv], buf, sem).wait()
        pltpu.sync_copy(buf, io_hbm.at[t, pl.ds(base, PPT)])


_BB = 1024
_G = BATCH // _BB


def _loss_body(u_r, i_r, y_r, out_r):
    g = pl.program_id(0)
    prod = u_r[...] * i_r[...]
    s1 = jnp.sum(prod, axis=2)
    pred = jnp.sum(s1, axis=0, keepdims=True)
    y = y_r[0]
    t = jnp.maximum(pred, 0.0) - pred * y + jnp.log1p(jnp.exp(-jnp.abs(pred)))
    part = jnp.sum(t)

    @pl.when(g == 0)
    def _():
        out_r[0, 0] = 0.0

    acc = out_r[0, 0] + part

    @pl.when(g < _G - 1)
    def _():
        out_r[0, 0] = acc

    @pl.when(g == _G - 1)
    def _():
        out_r[0, 0] = acc * (1.0 / BATCH)


_tc_loss = pl.pallas_call(
    _loss_body,
    grid=(_G,),
    in_specs=[
        pl.BlockSpec((NL + 1, _BB, D), lambda i: (0, i, 0)),
        pl.BlockSpec((NL + 1, _BB, D), lambda i: (0, i, 0)),
        pl.BlockSpec((1, 1, _BB), lambda i: (i, 0, 0)),
    ],
    out_specs=pl.BlockSpec(memory_space=pltpu.SMEM),
    out_shape=jax.ShapeDtypeStruct((1, 1), _f32),
)


def kernel(user_emb, item_emb, adj_vals, W_gc, b_gc, W_bi, b_bi,
           labels_list, user, item, edge_index, flag):
    ego0 = jnp.concatenate([user_emb[:-1], item_emb], axis=0)
    pad = EPAD - EDGES
    row = jnp.concatenate([edge_index[0], jnp.zeros((pad,), jnp.int32)])
    col = jnp.concatenate([edge_index[1], jnp.zeros((pad,), jnp.int32)])
    vals = jnp.concatenate([adj_vals, jnp.zeros((pad,), _f32)])
    zeros = jnp.zeros((NNP, D), _f32)

    ego = ego0
    norms = []
    for i in range(NL):
        parts = _sc_scatter(ego, col, row, vals, zeros)[:, :NN, :]
        ego, nrm = _tc_layer(parts, ego, W_gc[i], b_gc[i].reshape(1, D),
                             W_bi[i], b_bi[i].reshape(1, D))
        norms.append(nrm)

    u_all, i_all = _sc_gather(ego0, norms[0], norms[1], norms[2], user, item)
    loss = _tc_loss(u_all, i_all, labels_list.reshape(_G, 1, _BB))
    return loss[0, 0] + 0.0 * jnp.asarray(flag, _f32)


# speedup vs baseline: 1.0162x; 1.0162x over previous
import functools

import jax
import jax.numpy as jnp
from jax import lax
from jax.experimental import pallas as pl
from jax.experimental.pallas import tpu as pltpu
from jax.experimental.pallas import tpu_sc as plsc

NU = 5000
NN = 10000
NNP = 10112
EDGES = 320000
D = 128
NL = 3
BATCH = 4096

NC = 2
NS = 16
NW = NC * NS
CHUNK = 112
NBUF = 3
IBUF = 6
NCH0 = 126
NCH1 = 54
NCHT = NCH0 + NCH1
EPAD = NCHT * NS * CHUNK
_bf16 = jnp.bfloat16
PPT = BATCH // NW

_f32 = jnp.float32
_mesh = plsc.VectorSubcoreMesh(core_axis_name="c", subcore_axis_name="s")


@functools.partial(
    pl.kernel,
    out_type=jax.ShapeDtypeStruct((NC, NNP, D), _f32),
    mesh=_mesh,
    scratch_types=[
        pltpu.VMEM((IBUF, 2, CHUNK), jnp.int32),
        pltpu.VMEM((IBUF * CHUNK,), _f32),
        [pltpu.VMEM((CHUNK, D), _f32) for _ in range(NBUF)],
        pltpu.VMEM_SHARED((NNP, D), _f32),
        [pltpu.SemaphoreType.DMA for _ in range(NBUF)],
        [pltpu.SemaphoreType.DMA for _ in range(NBUF)],
        [pltpu.SemaphoreType.DMA for _ in range(IBUF)],
    ],
    compiler_params=pltpu.CompilerParams(needs_layout_passes=False),
)
def _sc_scatter(ego_hbm, col_hbm, row_hbm, val_hbm, z_hbm, out_hbm,
                ring, vring, bufs, side_sh, gsems, ssems, isems):
    c = lax.axis_index("c")
    s = lax.axis_index("s")
    nch = jnp.where(c == 0, NCH0, NCH1)
    cbase = jnp.where(c == 0, s * NCH0, NS * NCH0 + s * NCH1)
    stripe = NNP // NS
    pltpu.sync_copy(z_hbm.at[pl.ds(s * stripe, stripe)],
                    side_sh.at[pl.ds(s * stripe, stripe)])

    def idx_load_col(g, ib):
        return pltpu.make_async_copy(
            col_hbm.at[pl.ds((cbase + g) * CHUNK, CHUNK)],
            ring.at[ib, 0], isems[ib])

    def idx_load_row(g, ib):
        return pltpu.make_async_copy(
            row_hbm.at[pl.ds((cbase + g) * CHUNK, CHUNK)],
            ring.at[ib, 1], isems[ib])

    def idx_load_val(g, ib):
        return pltpu.make_async_copy(
            val_hbm.at[pl.ds((cbase + g) * CHUNK, CHUNK)],
            vring.at[pl.ds(ib * CHUNK, CHUNK)], isems[ib])

    def idx_start(g, ib):
        idx_load_col(g, ib).start()
        idx_load_row(g, ib).start()
        idx_load_val(g, ib).start()

    def idx_wait(g, ib):
        idx_load_col(g, ib).wait()
        idx_load_row(g, ib).wait()
        idx_load_val(g, ib).wait()

    def gather(g_dyn, b, ib):
        return pltpu.make_async_copy(ego_hbm.at[ring.at[ib, 0]], bufs[b],
                                     gsems[b])

    def scatter(g_dyn, b, ib):
        return pltpu.make_async_copy(bufs[b], side_sh.at[ring.at[ib, 1]],
                                     ssems[b])

    def scale(g, b, ib):
        buf = bufs[b]

        def group(q, cc):
            vv16 = vring[pl.ds(ib * CHUNK + q * 16, 16)]
            for j2 in range(16):
                vv = lax.gather(
                    vv16, jnp.full((16, 1), j2, jnp.int32),
                    lax.GatherDimensionNumbers(
                        offset_dims=(), collapsed_slice_dims=(0,),
                        start_index_map=(0,)),
                    slice_sizes=(1,),
                    mode=lax.GatherScatterMode.PROMISE_IN_BOUNDS)
                for r in range(D // 16):
                    sl = (q * 16 + j2, pl.ds(r * 16, 16))
                    buf[sl] = buf[sl] * vv
            return cc

        lax.fori_loop(0, CHUNK // 16, group, 0)

    for k in range(4):
        idx_start(k, k)
    idx_wait(0, 0)
    gather(0, 0, 0).start()

    def pipe(h, carry):
        g0 = h * IBUF
        for j in range(IBUF):
            g = g0 + j
            b = j % NBUF
            nb = (j + 1) % NBUF
            nib = (j + 1) % IBUF
            gather(g, b, j).wait()

            @pl.when(g >= 2)
            def _():
                scatter(g - 2, nb, nib).wait()

            @pl.when(g + 1 < nch)
            def _():
                idx_wait(g + 1, nib)
                gather(g + 1, nb, nib).start()

            scale(g, b, j)
            scatter(g, b, j).start(add=True)

            @pl.when(g + 4 < nch)
            def _():
                idx_start(g + 4, (j + 4) % IBUF)
        return carry

    lax.fori_loop(0, nch // IBUF, pipe, 0)
    scatter(0, (-2) % NBUF, (-2) % IBUF).wait()
    scatter(0, (-1) % NBUF, (-1) % IBUF).wait()
    plsc.subcore_barrier()
    pltpu.sync_copy(side_sh.at[pl.ds(s * stripe, stripe)],
                    out_hbm.at[c, pl.ds(s * stripe, stripe)])


_BLK = 2000


def _layer_body(parts_r, ego_r, wg_r, bg_r, wb_r, bb_r, ego_o, norm_o):
    side = parts_r[0] + parts_r[1]
    e = ego_r[...]
    x = jnp.dot(side, wg_r[...], preferred_element_type=_f32) + bg_r[...]
    se = jnp.where(x >= 0, x, 0.01 * x)
    y = jnp.dot(e * side, wb_r[...], preferred_element_type=_f32) + bb_r[...]
    bi = jnp.where(y >= 0, y, 0.01 * y)
    new = se + bi
    nrm = jnp.sqrt(jnp.sum(new * new, axis=1, keepdims=True))
    ego_o[...] = new
    norm_o[...] = new / jnp.maximum(nrm, 1e-12)


_tc_layer = pl.pallas_call(
    _layer_body,
    grid=(NN // _BLK,),
    in_specs=[
        pl.BlockSpec((NC, _BLK, D), lambda i: (0, i, 0)),
        pl.BlockSpec((_BLK, D), lambda i: (i, 0)),
        pl.BlockSpec((D, D), lambda i: (0, 0)),
        pl.BlockSpec((1, D), lambda i: (0, 0)),
        pl.BlockSpec((D, D), lambda i: (0, 0)),
        pl.BlockSpec((1, D), lambda i: (0, 0)),
    ],
    out_specs=[pl.BlockSpec((_BLK, D), lambda i: (i, 0))] * 2,
    out_shape=[jax.ShapeDtypeStruct((NN, D), _f32)] * 2,
)


@functools.partial(
    pl.kernel,
    out_type=(jax.ShapeDtypeStruct((NL + 1, BATCH, D), _f32),
              jax.ShapeDtypeStruct((NL + 1, BATCH, D), _f32)),
    mesh=_mesh,
    scratch_types=[
        pltpu.VMEM((PPT,), jnp.int32),
        pltpu.VMEM((PPT,), jnp.int32),
        pltpu.VMEM((PPT, D), _f32),
        pltpu.SemaphoreType.DMA,
    ],
)
def _sc_gather(t0, t1, t2, t3, u_hbm, it_hbm, uo_hbm, io_hbm,
               uv, iv, buf, sem):
    c = lax.axis_index("c")
    s = lax.axis_index("s")
    base = (c * NS + s) * PPT
    pltpu.sync_copy(u_hbm.at[pl.ds(base, PPT)], uv)
    pltpu.sync_copy(it_hbm.at[pl.ds(base, PPT)], iv)
    for r in range(PPT // 16):
        iv[pl.ds(r * 16, 16)] = iv[pl.ds(r * 16, 16)] + NU
    for t, tbl in enumerate((t0, t1, t2, t3)):
        pltpu.async_copy(tbl.at[uv], buf, sem).wait()
        pltpu.sync_copy(buf, uo_hbm.at[t, pl.ds(base, PPT)])
        pltpu.async_copy(tbl.at[iv], buf, sem).wait()
        pltpu.sync_copy(buf, io_hbm.at[t, pl.ds(base, PPT)])


_BB = 1024
_G = BATCH // _BB


def _loss_body(u_r, i_r, y_r, out_r):
    g = pl.program_id(0)
    prod = u_r[...] * i_r[...]
    s1 = jnp.sum(prod, axis=2)
    pred = jnp.sum(s1, axis=0, keepdims=True)
    y = y_r[0]
    t = jnp.maximum(pred, 0.0) - pred * y + jnp.log1p(jnp.exp(-jnp.abs(pred)))
    part = jnp.sum(t)

    @pl.when(g == 0)
    def _():
        out_r[0, 0] = 0.0

    acc = out_r[0, 0] + part

    @pl.when(g < _G - 1)
    def _():
        out_r[0, 0] = acc

    @pl.when(g == _G - 1)
    def _():
        out_r[0, 0] = acc * (1.0 / BATCH)


_tc_loss = pl.pallas_call(
    _loss_body,
    grid=(_G,),
    in_specs=[
        pl.BlockSpec((NL + 1, _BB, D), lambda i: (0, i, 0)),
        pl.BlockSpec((NL + 1, _BB, D), lambda i: (0, i, 0)),
        pl.BlockSpec((1, 1, _BB), lambda i: (i, 0, 0)),
    ],
    out_specs=pl.BlockSpec(memory_space=pltpu.SMEM),
    out_shape=jax.ShapeDtypeStruct((1, 1), _f32),
)


def kernel(user_emb, item_emb, adj_vals, W_gc, b_gc, W_bi, b_bi,
           labels_list, user, item, edge_index, flag):
    ego0 = jnp.concatenate([user_emb[:-1], item_emb], axis=0)
    pad = EPAD - EDGES
    row = jnp.concatenate([edge_index[0], jnp.zeros((pad,), jnp.int32)])
    col = jnp.concatenate([edge_index[1], jnp.zeros((pad,), jnp.int32)])
    vals = jnp.concatenate([adj_vals, jnp.zeros((pad,), _f32)])
    zeros = jnp.zeros((NNP, D), _f32)

    ego = ego0
    norms = []
    for i in range(NL):
        parts = _sc_scatter(ego, col, row, vals, zeros)[:, :NN, :]
        ego, nrm = _tc_layer(parts, ego, W_gc[i], b_gc[i].reshape(1, D),
                             W_bi[i], b_bi[i].reshape(1, D))
        norms.append(nrm)

    u_all, i_all = _sc_gather(ego0, norms[0], norms[1], norms[2], user, item)
    loss = _tc_loss(u_all, i_all, labels_list.reshape(_G, 1, _BB))
    return loss[0, 0] + 0.0 * jnp.asarray(flag, _f32)
